# baseline (speedup 1.0000x reference)
import jax
import jax.numpy as jnp
from jax.experimental import pallas as pl
from jax.experimental.pallas import tpu as pltpu

N_CAM = 8
SEG = 12500
CHUNK = 2500
K_CHUNKS = SEG // CHUNK
B = 64
D = 128
BETA = 0.05


def _loss_kernel(x_ref, em_ref, lab_ref, out_ref,
                 m_ref, s_ref, dot_ref, lsum_ref, lmax_ref):
    c = pl.program_id(0)
    k = pl.program_id(1)

    @pl.when(jnp.logical_and(c == 0, k == 0))
    def _init_loss():
        out_ref[...] = jnp.zeros((1, 1), jnp.float32)

    @pl.when(k == 0)
    def _init_seg():
        m_ref[...] = jnp.full((B, 1), -1e30, jnp.float32)
        s_ref[...] = jnp.zeros((B, 1), jnp.float32)
        dot_ref[...] = jnp.zeros((B, D), jnp.float32)
        lsum_ref[...] = jnp.zeros((B, 1), jnp.float32)
        lmax_ref[...] = jnp.full((B, 1), -1e30, jnp.float32)

    x = x_ref[...]
    xn = x / jnp.maximum(
        jnp.sqrt(jnp.sum(x * x, axis=1, keepdims=True)), 1e-12)

    em = em_ref[0]
    lab = lab_ref[:, 0, 0, :]

    logits = jax.lax.dot_general(
        xn, em, (((1,), (1,)), ((), ())),
        preferred_element_type=jnp.float32) * (1.0 / BETA)

    bm = jnp.max(logits, axis=1, keepdims=True)
    m_old = m_ref[...]
    m_new = jnp.maximum(m_old, bm)
    s_ref[...] = (s_ref[...] * jnp.exp(m_old - m_new)
                  + jnp.sum(jnp.exp(logits - m_new), axis=1, keepdims=True))
    m_ref[...] = m_new

    dot_ref[...] += jnp.dot(lab, em, preferred_element_type=jnp.float32)
    lsum_ref[...] += jnp.sum(lab, axis=1, keepdims=True)
    lmax_ref[...] = jnp.maximum(lmax_ref[...],
                                jnp.max(lab, axis=1, keepdims=True))

    @pl.when(k == K_CHUNKS - 1)
    def _finalize_seg():
        lse = m_ref[...] + jnp.log(s_ref[...])
        rowdot = jnp.sum(xn * dot_ref[...], axis=1,
                         keepdims=True) * (1.0 / BETA)
        v = (lsum_ref[...] * lse - rowdot) / (lmax_ref[...] + 1e-20)
        out_ref[...] += jnp.sum(v, axis=0, keepdims=True) / (B * N_CAM)


def kernel(x, pids, img_index, cams, labels, em_all):
    em_r = em_all.reshape(N_CAM * K_CHUNKS, CHUNK, D)
    lab_r = labels.reshape(B, N_CAM * K_CHUNKS, 1, CHUNK)

    out = pl.pallas_call(
        _loss_kernel,
        grid=(N_CAM, K_CHUNKS),
        in_specs=[
            pl.BlockSpec((B, D), lambda c, k: (0, 0)),
            pl.BlockSpec((1, CHUNK, D), lambda c, k: (c * K_CHUNKS + k, 0, 0)),
            pl.BlockSpec((B, 1, 1, CHUNK),
                         lambda c, k: (0, c * K_CHUNKS + k, 0, 0)),
        ],
        out_specs=pl.BlockSpec((1, 1), lambda c, k: (0, 0)),
        out_shape=jax.ShapeDtypeStruct((1, 1), jnp.float32),
        scratch_shapes=[
            pltpu.VMEM((B, 1), jnp.float32),
            pltpu.VMEM((B, 1), jnp.float32),
            pltpu.VMEM((B, D), jnp.float32),
            pltpu.VMEM((B, 1), jnp.float32),
            pltpu.VMEM((B, 1), jnp.float32),
        ],
    )(x, em_r, lab_r)
    return out.reshape(())


# speedup vs baseline: 1.0773x; 1.0773x over previous
import jax
import jax.numpy as jnp
from jax.experimental import pallas as pl
from jax.experimental.pallas import tpu as pltpu

N_CAM = 8
SEG = 12500
SUB = 2048
B = 64
D = 128
BETA = 0.05

_OFFS = [(o, min(SUB, SEG - o)) for o in range(0, SEG, SUB)]


def _loss_kernel(x_ref, em_ref, lab_ref, out_ref):
    c = pl.program_id(0)

    @pl.when(c == 0)
    def _init_loss():
        out_ref[...] = jnp.zeros((1, 1), jnp.float32)

    x = x_ref[...]
    xn = x / jnp.maximum(
        jnp.sqrt(jnp.sum(x * x, axis=1, keepdims=True)), 1e-12)

    m = jnp.full((B, 1), -1e30, jnp.float32)
    s = jnp.zeros((B, 1), jnp.float32)
    dotacc = jnp.zeros((B, D), jnp.float32)
    lsum = jnp.zeros((B, 1), jnp.float32)
    lmax = jnp.full((B, 1), -1e30, jnp.float32)

    for off, sz in _OFFS:
        em = em_ref[0, off:off + sz, :]
        lab = lab_ref[:, 0, 0, off:off + sz]

        logits = jax.lax.dot_general(
            xn, em, (((1,), (1,)), ((), ())),
            preferred_element_type=jnp.float32) * (1.0 / BETA)

        bm = jnp.max(logits, axis=1, keepdims=True)
        m_new = jnp.maximum(m, bm)
        s = (s * jnp.exp(m - m_new)
             + jnp.sum(jnp.exp(logits - m_new), axis=1, keepdims=True))
        m = m_new

        dotacc = dotacc + jnp.dot(lab, em, preferred_element_type=jnp.float32)
        lsum = lsum + jnp.sum(lab, axis=1, keepdims=True)
        lmax = jnp.maximum(lmax, jnp.max(lab, axis=1, keepdims=True))

    lse = m + jnp.log(s)
    rowdot = jnp.sum(xn * dotacc, axis=1,
                     keepdims=True) * (1.0 / BETA)
    v = (lsum * lse - rowdot) / (lmax + 1e-20)
    out_ref[...] += jnp.sum(v, axis=0, keepdims=True) / (B * N_CAM)


def kernel(x, pids, img_index, cams, labels, em_all):
    em_r = em_all.reshape(N_CAM, SEG, D)
    lab_r = labels.reshape(B, N_CAM, 1, SEG)

    out = pl.pallas_call(
        _loss_kernel,
        grid=(N_CAM,),
        in_specs=[
            pl.BlockSpec((B, D), lambda c: (0, 0)),
            pl.BlockSpec((1, SEG, D), lambda c: (c, 0, 0)),
            pl.BlockSpec((B, 1, 1, SEG), lambda c: (0, c, 0, 0)),
        ],
        out_specs=pl.BlockSpec((1, 1), lambda c: (0, 0)),
        out_shape=jax.ShapeDtypeStruct((1, 1), jnp.float32),
    )(x, em_r, lab_r)
    return out.reshape(())
